# Initial kernel scaffold; baseline (speedup 1.0000x reference)
import jax
import jax.numpy as jnp
from jax.experimental import pallas as pl


def kernel(z, pos, edge_index, batch, params):
    raise NotImplementedError("write your pallas kernel here")


# speedup vs baseline: 2.1301x; 2.1301x over previous
import functools

import jax
import jax.numpy as jnp
from jax import lax
from jax.experimental import pallas as pl
from jax.experimental.pallas import tpu as pltpu, tpu_sc as plsc

F32 = jnp.float32
I32 = jnp.int32


N = 10000
E = 320000
H = 128
NGRAPH = 256

NP = N + 16
NW = 32
CH = 128
EW = 10240
EP = NW * EW
NCH = EW // CH

BN = 1000
BE = 1024


def _sigm(x):
    return 1.0 / (1.0 + jnp.exp(-x))


def _silu(x):
    return x * _sigm(x)


def _embed_body(z_ref, emb_ref, o_ref):
    zb = z_ref[:, 0]
    oh = (zb[:, None] == lax.broadcasted_iota(I32, (BN, 128), 1)).astype(F32)
    o_ref[...] = jnp.dot(oh, emb_ref[...], preferred_element_type=F32, precision=lax.Precision.HIGHEST)


def _tables_body(h_ref, wsd_ref, t1_ref, t2_ref):
    hb = h_ref[...]
    t1_ref[...] = jnp.dot(hb, wsd_ref[0], preferred_element_type=F32)
    t2_ref[...] = jnp.dot(hb, wsd_ref[1], preferred_element_type=F32)


def _edge_body(g1_ref, g2_ref, rd_ref, w2_ref, pe_ref, m2_ref, wrs_ref):
    rd = rd_ref[...]
    lane = lax.broadcasted_iota(I32, (BE, 16), 1)
    rel = jnp.where(lane < 3, rd, 0.0)
    dist2 = rd[:, 3:4]
    pe = pe_ref[...]
    d2b = dist2.astype(jnp.bfloat16).astype(F32)
    x = g1_ref[...] + g2_ref[...] + d2b * pe[1][None, :] + pe[0][None, :]
    m = _silu(x)
    y = jnp.dot(m, w2_ref[...], preferred_element_type=F32) + pe[2][None, :]
    m2 = _silu(y)
    m2b = m2.astype(jnp.bfloat16).astype(F32)
    w = jnp.sum(m2b * pe[3][None, :], axis=1, keepdims=True) + pe[4:5, :1]
    m2_ref[...] = m2
    wr = jnp.where(lane == 3, 1.0, rel * w)
    ey = (lax.broadcasted_iota(I32, (16, 16), 0)
          == lax.broadcasted_iota(I32, (16, 16), 1)).astype(F32)
    wrs_ref[...] = lax.dot_general(ey, wr, (((1,), (1,)), ((), ())),
                                   preferred_element_type=F32,
                                   precision=lax.Precision.HIGHEST)


def _node_body(first, h_ref, a0_ref, a1_ref, cp_ref, p16_ref, dg_ref,
               wn_ref, pb_ref, ho_ref, po_ref, dgo_ref):
    hb = h_ref[...]
    agg = a0_ref[0] + a1_ref[0]
    x = (jnp.dot(hb, wn_ref[0], preferred_element_type=F32)
         + jnp.dot(agg, wn_ref[1], preferred_element_type=F32)
         + pb_ref[0][None, :])
    t = _silu(x)
    ho_ref[...] = hb + jnp.dot(t, wn_ref[2], preferred_element_type=F32) + pb_ref[1][None, :]
    crd4 = jnp.sum(cp_ref[...], axis=0)
    crd = jnp.concatenate([crd4, jnp.zeros((BN, 12), F32)], axis=1)
    if first:
        d = crd[:, 3:4] + 1.0
    else:
        d = dg_ref[:, 3:4]
    lane = lax.broadcasted_iota(I32, (BN, 16), 1)
    po_ref[...] = p16_ref[...] + jnp.where(lane < 3, crd * (1.0 / d), 0.0)
    dgo_ref[...] = jnp.broadcast_to(d, (BN, 16))


def _pool_body(b_ref, h_ref, o1_ref, ph_ref, out_ref, acc_ref):
    i = pl.program_id(0)

    @pl.when(i == 0)
    def _():
        acc_ref[...] = jnp.zeros((NGRAPH, 128), F32)

    bb = b_ref[:, 0]
    oh = (bb[:, None] == lax.broadcasted_iota(I32, (BN, NGRAPH), 1)).astype(F32)
    acc_ref[...] += lax.dot_general(oh, h_ref[...], (((0,), (0,)), ((), ())),
                                    preferred_element_type=F32, precision=lax.Precision.HIGHEST)
    hg = acc_ref[...]
    xg = jnp.dot(hg, o1_ref[...], preferred_element_type=F32) + ph_ref[0][None, :]
    og = _silu(xg).astype(jnp.bfloat16).astype(F32)
    out_ref[...] = jnp.sum(og * ph_ref[1][None, :], axis=1, keepdims=True) + ph_ref[2:3, :1]


def _sc_mesh():
    return plsc.VectorSubcoreMesh(core_axis_name="c", subcore_axis_name="s")


def _gather_body(t1_hbm, t2_hbm, src_hbm, dst_hbm, px_hbm, py_hbm, pz_hbm,
                 z16_hbm, g1_hbm, g2_hbm, rd_hbm,
                 idx1_v, idx2_v, rows1_v, rows2_v, px_v, py_v, pz_v, rd_v,
                 sem1, sem2):
    c = lax.axis_index("c")
    s = lax.axis_index("s")
    base = (s * 2 + c) * EW

    pltpu.sync_copy(px_hbm, px_v)
    pltpu.sync_copy(py_hbm, py_v)
    pltpu.sync_copy(pz_hbm, pz_v)
    pltpu.sync_copy(z16_hbm.at[pl.ds(0, CH)], rd_v)

    lane16 = lax.broadcasted_iota(I32, (16,), 0)

    def body(g, carry):
        off = base + g * CH
        pltpu.sync_copy(src_hbm.at[pl.ds(off, CH)], idx1_v)
        pltpu.sync_copy(dst_hbm.at[pl.ds(off, CH)], idx2_v)
        cp1 = pltpu.make_async_copy(t1_hbm.at[idx1_v], rows1_v, sem1)
        cp2 = pltpu.make_async_copy(t2_hbm.at[idx2_v], rows2_v, sem2)
        cp1.start()
        cp2.start()
        for k in range(CH // 16):
            iv1 = idx1_v[pl.ds(k * 16, 16)]
            iv2 = idx2_v[pl.ds(k * 16, 16)]
            rx = plsc.load_gather(px_v, [iv1]) - plsc.load_gather(px_v, [iv2])
            ry = plsc.load_gather(py_v, [iv1]) - plsc.load_gather(py_v, [iv2])
            rz = plsc.load_gather(pz_v, [iv1]) - plsc.load_gather(pz_v, [iv2])
            d2 = rx * rx + ry * ry + rz * rz
            rows = k * 16 + lane16
            plsc.store_scatter(rd_v, [rows, jnp.full((16,), 0, I32)], rx)
            plsc.store_scatter(rd_v, [rows, jnp.full((16,), 1, I32)], ry)
            plsc.store_scatter(rd_v, [rows, jnp.full((16,), 2, I32)], rz)
            plsc.store_scatter(rd_v, [rows, jnp.full((16,), 3, I32)], d2)
        cp1.wait()
        cp2.wait()
        pltpu.sync_copy(rows1_v, g1_hbm.at[pl.ds(off, CH)])
        pltpu.sync_copy(rows2_v, g2_hbm.at[pl.ds(off, CH)])
        pltpu.sync_copy(rd_v, rd_hbm.at[pl.ds(off, CH)])
        return carry

    lax.fori_loop(0, NCH, body, 0)


def _scatter_body(m2_hbm, dst_hbm, z128_hbm, agg_hbm, sh128, idx_v, val_v):
    c = lax.axis_index("c")
    s = lax.axis_index("s")
    base = (s * 2 + c) * EW

    @pl.when(s == 0)
    def _():
        pltpu.sync_copy(z128_hbm, sh128)

    plsc.subcore_barrier()

    def body(g, carry):
        off = base + g * CH
        pltpu.sync_copy(dst_hbm.at[pl.ds(off, CH)], idx_v)
        pltpu.sync_copy(m2_hbm.at[pl.ds(off, CH)], val_v)
        pltpu.sync_copy(val_v, sh128.at[idx_v], add=True)
        return carry

    lax.fori_loop(0, NCH, body, 0)
    plsc.subcore_barrier()

    @pl.when(s == 0)
    def _():
        pltpu.sync_copy(sh128, agg_hbm.at[c])


def _coord_body(wrs_hbm, dst_hbm, z4_hbm, dep_hbm, crd_hbm, acc4_v, idx_v, valc_v):
    del dep_hbm
    c = lax.axis_index("c")
    s = lax.axis_index("s")
    wid = s * 2 + c
    base = wid * EW

    pltpu.sync_copy(z4_hbm, acc4_v)

    def body(g, carry):
        off = base + g * CH
        pltpu.sync_copy(dst_hbm.at[pl.ds(off, CH)], idx_v)
        pltpu.sync_copy(wrs_hbm.at[:, pl.ds(off, CH)], valc_v)
        for k in range(CH // 16):
            idxv = idx_v[pl.ds(k * 16, 16)]
            for j in range(4):
                vals = valc_v[j, pl.ds(k * 16, 16)]
                plsc.addupdate_scatter(acc4_v, [jnp.full((16,), j, I32), idxv], vals)
        return carry

    lax.fori_loop(0, NCH, body, 0)
    pltpu.sync_copy(acc4_v, crd_hbm.at[wid])


def kernel(z, pos, edge_index, batch, params):
    pos3 = pos[:, 2, :]
    pos16 = jnp.pad(pos3, ((0, 0), (0, 13)))
    src = edge_index[0].astype(I32)
    dst = edge_index[1].astype(I32)
    npad = EP - E
    srcp = jnp.concatenate([src, jnp.zeros((npad,), I32)])
    dstg = jnp.concatenate([dst, jnp.zeros((npad,), I32)])
    dsts = jnp.concatenate([dst, jnp.full((npad,), N, I32)])
    z128 = jnp.zeros((NP, 128), F32)
    z16 = jnp.zeros((NP, 16), F32)
    z4 = jnp.zeros((4, NP), F32)

    embed_p = jnp.pad(params["embed"], ((0, 128 - params["embed"].shape[0]), (0, 0)))
    z2 = z.astype(I32)[:, None]
    batch2 = batch.astype(I32)[:, None]

    grid_n = N // BN

    h = pl.pallas_call(
        _embed_body,
        grid=(grid_n,),
        in_specs=[pl.BlockSpec((BN, 1), lambda i: (i, 0)),
                  pl.BlockSpec((128, 128), lambda i: (0, 0))],
        out_specs=pl.BlockSpec((BN, 128), lambda i: (i, 0)),
        out_shape=jax.ShapeDtypeStruct((N, 128), F32),
    )(z2, embed_p)

    tables_call = pl.pallas_call(
        _tables_body,
        grid=(grid_n,),
        in_specs=[pl.BlockSpec((BN, 128), lambda i: (i, 0)),
                  pl.BlockSpec((2, 128, 128), lambda i: (0, 0, 0))],
        out_specs=[pl.BlockSpec((BN, 128), lambda i: (i, 0)),
                   pl.BlockSpec((BN, 128), lambda i: (i, 0))],
        out_shape=[jax.ShapeDtypeStruct((N, 128), F32),
                   jax.ShapeDtypeStruct((N, 128), F32)],
    )

    gather_call = functools.partial(
        pl.kernel,
        out_type=[jax.ShapeDtypeStruct((EP, 128), F32),
                  jax.ShapeDtypeStruct((EP, 128), F32),
                  jax.ShapeDtypeStruct((EP, 16), F32)],
        mesh=_sc_mesh(),
        scratch_types=[pltpu.VMEM((CH,), I32), pltpu.VMEM((CH,), I32),
                       pltpu.VMEM((CH, 128), F32), pltpu.VMEM((CH, 128), F32),
                       pltpu.VMEM((N,), F32), pltpu.VMEM((N,), F32),
                       pltpu.VMEM((N,), F32), pltpu.VMEM((CH, 16), F32),
                       pltpu.SemaphoreType.DMA, pltpu.SemaphoreType.DMA],
        compiler_params=pltpu.CompilerParams(needs_layout_passes=False),
    )(_gather_body)

    edge_call = pl.pallas_call(
        _edge_body,
        grid=(EP // BE,),
        in_specs=[pl.BlockSpec((BE, 128), lambda i: (i, 0)),
                  pl.BlockSpec((BE, 128), lambda i: (i, 0)),
                  pl.BlockSpec((BE, 16), lambda i: (i, 0)),
                  pl.BlockSpec((128, 128), lambda i: (0, 0)),
                  pl.BlockSpec((8, 128), lambda i: (0, 0))],
        out_specs=[pl.BlockSpec((BE, 128), lambda i: (i, 0)),
                   pl.BlockSpec((16, BE), lambda i: (0, i))],
        out_shape=[jax.ShapeDtypeStruct((EP, 128), F32),
                   jax.ShapeDtypeStruct((16, EP), F32)],
    )

    scatter_call = functools.partial(
        pl.kernel,
        out_type=jax.ShapeDtypeStruct((2, NP, 128), F32),
        mesh=_sc_mesh(),
        scratch_types=[pltpu.VMEM_SHARED((NP, 128), F32),
                       pltpu.VMEM((CH,), I32),
                       pltpu.VMEM((CH, 128), F32)],
    )(_scatter_body)

    coord_call = functools.partial(
        pl.kernel,
        out_type=jax.ShapeDtypeStruct((NW, 4, NP), F32),
        mesh=_sc_mesh(),
        scratch_types=[pltpu.VMEM((4, NP), F32),
                       pltpu.VMEM((CH,), I32),
                       pltpu.VMEM((16, CH), F32)],
        compiler_params=pltpu.CompilerParams(needs_layout_passes=False),
    )(_coord_body)

    def node_call(first):
        return pl.pallas_call(
            functools.partial(_node_body, first),
            grid=(grid_n,),
            in_specs=[pl.BlockSpec((BN, 128), lambda i: (i, 0)),
                      pl.BlockSpec((1, BN, 128), lambda i: (0, i, 0)),
                      pl.BlockSpec((1, BN, 128), lambda i: (1, i, 0)),
                      pl.BlockSpec((NW, BN, 4), lambda i: (0, i, 0)),
                      pl.BlockSpec((BN, 16), lambda i: (i, 0)),
                      pl.BlockSpec((BN, 16), lambda i: (i, 0)),
                      pl.BlockSpec((3, 128, 128), lambda i: (0, 0, 0)),
                      pl.BlockSpec((8, 128), lambda i: (0, 0))],
            out_specs=[pl.BlockSpec((BN, 128), lambda i: (i, 0)),
                       pl.BlockSpec((BN, 16), lambda i: (i, 0)),
                       pl.BlockSpec((BN, 16), lambda i: (i, 0))],
            out_shape=[jax.ShapeDtypeStruct((N, 128), F32),
                       jax.ShapeDtypeStruct((N, 16), F32),
                       jax.ShapeDtypeStruct((N, 16), F32)],
        )

    deg16 = jnp.zeros((N, 16), F32)
    for li, layer in enumerate(params["layers"]):
        w1 = layer["edge1"]["W"]
        wsd = jnp.stack([w1[:H], w1[H:2 * H]])
        pe = jnp.zeros((8, 128), F32)
        pe = pe.at[0].set(layer["edge1"]["b"])
        pe = pe.at[1].set(w1[2 * H].astype(jnp.bfloat16).astype(F32))
        pe = pe.at[2].set(layer["edge2"]["b"])
        pe = pe.at[3].set(layer["coord"]["W"][:, 0].astype(jnp.bfloat16).astype(F32))
        pe = pe.at[4].set(jnp.full((128,), layer["coord"]["b"][0]))
        wn1 = layer["node1"]["W"]
        wn = jnp.stack([wn1[:H], wn1[H:], layer["node2"]["W"]])
        pb = jnp.zeros((8, 128), F32)
        pb = pb.at[0].set(layer["node1"]["b"])
        pb = pb.at[1].set(layer["node2"]["b"])

        px = pos16[:, 0]
        py = pos16[:, 1]
        pz = pos16[:, 2]
        t1, t2 = tables_call(h, wsd)
        g1, g2, rd = gather_call(t1, t2, srcp, dstg, px, py, pz, z16)
        m2, wrs = edge_call(g1, g2, rd, layer["edge2"]["W"], pe)
        aggp = scatter_call(m2, dsts, z128)
        crdp = jnp.transpose(coord_call(wrs, dsts, z4, aggp), (0, 2, 1))
        h, pos16, deg16 = node_call(li == 0)(
            h, aggp, aggp, crdp, pos16, deg16, wn, pb)

    ph = jnp.zeros((8, 128), F32)
    ph = ph.at[0].set(params["out1"]["b"])
    ph = ph.at[1].set(params["out2"]["W"][:, 0].astype(jnp.bfloat16).astype(F32))
    ph = ph.at[2].set(jnp.full((128,), params["out2"]["b"][0]))

    out = pl.pallas_call(
        _pool_body,
        grid=(grid_n,),
        in_specs=[pl.BlockSpec((BN, 1), lambda i: (i, 0)),
                  pl.BlockSpec((BN, 128), lambda i: (i, 0)),
                  pl.BlockSpec((128, 128), lambda i: (0, 0)),
                  pl.BlockSpec((8, 128), lambda i: (0, 0))],
        out_specs=pl.BlockSpec((NGRAPH, 1), lambda i: (0, 0)),
        out_shape=jax.ShapeDtypeStruct((NGRAPH, 1), F32),
        scratch_shapes=[pltpu.VMEM((NGRAPH, 128), F32)],
    )(batch2, h, params["out1"]["W"], ph)

    return out
